# baseline (speedup 1.0000x reference)
import dataclasses
import functools

import jax
import jax.numpy as jnp
from jax import lax
from jax.experimental import pallas as pl
from jax.experimental.pallas import tpu as pltpu
from jax.experimental.pallas import tpu_sc as plsc


def _sc_params(**kw):
    cp = pltpu.CompilerParams(**kw)
    if "needs_layout_passes" in pltpu.CompilerParams.__dataclass_fields__:
        cp = dataclasses.replace(cp, needs_layout_passes=False)
    return cp

NC = 2
NS = 16
NW = NC * NS
LANES = 16

CB = 16


def _sc_pack(user_table, ad_table):
    Vu, E = user_table.shape
    Va = ad_table.shape[0]
    NBu = Vu // 128
    NBa = Va // 128
    TU = 2 * ((NBu // NW + 2) // 2)
    TA = 2 * ((NBa // NW + 2) // 2)

    ut_t = user_table.T
    at_t = ad_table.T
    ut_last = user_table[Vu - 128:].T
    at_last = ad_table[Va - 128:].T

    mesh = plsc.VectorSubcoreMesh(core_axis_name="c", subcore_axis_name="s")

    @functools.partial(
        pl.kernel,
        mesh=mesh,
        compiler_params=_sc_params(use_tc_tiling_on_sc=True),
        out_type=(jax.ShapeDtypeStruct((Vu // 2, 128), jnp.float32),
                  jax.ShapeDtypeStruct((Va // 2, 128), jnp.float32)),
        scratch_types=[
            pltpu.VMEM((E, 128), jnp.float32),
            pltpu.VMEM((E, 128), jnp.float32),
            pltpu.VMEM((64, 128), jnp.float32),
            pltpu.VMEM((64, 128), jnp.float32),
            pltpu.SemaphoreType.DMA,
            pltpu.SemaphoreType.DMA,
            pltpu.SemaphoreType.DMA,
            pltpu.SemaphoreType.DMA,
        ],
    )
    def pack_kernel(ut_hbm, at_hbm, ulast_hbm, alast_hbm, ou_hbm, oa_hbm,
                    tin0, tin1, ob0, ob1, si0, si1, so0, so1):
        wid = lax.axis_index("s") * NC + lax.axis_index("c")

        iota = lax.iota(jnp.int32, 16)
        half = iota >> 1
        par64 = (iota & 1) * 64
        row_base = [half + g * 8 for g in range(8)]

        def transpose(tin, ob):
            @pl.loop(0, E)
            def _(j):
                col = par64 + j
                for g in range(8):
                    v = tin[j, pl.ds(g * 16, 16)]
                    plsc.store_scatter(ob, [row_base[g], col], v)

        def run_table(src_hbm, out_hbm, nb, t_slots):
            def kb_of(t):
                return t * NW + wid

            def start_in(t, tin, sem):
                @pl.when(kb_of(t) < nb)
                def _():
                    pltpu.async_copy(
                        src_hbm.at[:, pl.ds(kb_of(t) * 128, 128)], tin, sem)

            def body(g, t, tin, ob, sem_i, sem_o):
                @pl.when(kb_of(t) < nb)
                def _():
                    pltpu.make_async_copy(
                        src_hbm.at[:, pl.ds(0, 128)], tin, sem_i).wait()

                    @pl.when(g >= 2)
                    def _():
                        pltpu.make_async_copy(
                            out_hbm.at[pl.ds(0, 64)], ob, sem_o).wait()

                    transpose(tin, ob)
                    pltpu.async_copy(
                        ob, out_hbm.at[pl.ds(kb_of(t) * 64, 64)], sem_o)

            start_in(0, tin0, si0)

            @pl.loop(0, t_slots, step=2)
            def _(g):
                start_in(g + 1, tin1, si1)
                body(g, g, tin0, ob0, si0, so0)
                start_in(g + 2, tin0, si0)
                body(g, g + 1, tin1, ob1, si1, so1)

            pltpu.make_async_copy(out_hbm.at[pl.ds(0, 64)], ob0, so0).wait()
            pltpu.make_async_copy(out_hbm.at[pl.ds(0, 64)], ob1, so1).wait()

        run_table(ut_hbm, ou_hbm, NBu, TU)
        run_table(at_hbm, oa_hbm, NBa, TA)

        @pl.when(wid == 4)
        def _():
            pltpu.sync_copy(ulast_hbm, tin0)
            transpose(tin0, ob0)
            pltpu.sync_copy(ob0, ou_hbm.at[pl.ds((Vu - 128) // 2, 64)])

        @pl.when(wid == 13)
        def _():
            pltpu.sync_copy(alast_hbm, tin1)
            transpose(tin1, ob1)
            pltpu.sync_copy(ob1, oa_hbm.at[pl.ds((Va - 128) // 2, 64)])

    ou, oa = pack_kernel(ut_t, at_t, ut_last, at_last)
    return ou.reshape(Vu, E), oa.reshape(Va, E)


def _sc_dot(uid_flat, aid_flat, u_lin, a_lin, B, L, E):
    IDX = CB * L
    rows_per_w = B // NW
    chunks = rows_per_w // CB
    idx_per_w = rows_per_w * L
    assert rows_per_w % CB == 0 and chunks % 2 == 0
    n_lane_grp = E // LANES

    mesh = plsc.VectorSubcoreMesh(core_axis_name="c", subcore_axis_name="s")

    @functools.partial(
        pl.kernel,
        mesh=mesh,
        compiler_params=pltpu.CompilerParams(use_tc_tiling_on_sc=False),
        out_type=jax.ShapeDtypeStruct((B, E), jnp.float32),
        scratch_types=[
            pltpu.VMEM((idx_per_w,), jnp.int32),
            pltpu.VMEM((idx_per_w,), jnp.int32),
            pltpu.VMEM((IDX, E), jnp.float32),
            pltpu.VMEM((IDX, E), jnp.float32),
            pltpu.VMEM((IDX, E), jnp.float32),
            pltpu.VMEM((IDX, E), jnp.float32),
            pltpu.VMEM((CB, E), jnp.float32),
            pltpu.VMEM((CB, E), jnp.float32),
            pltpu.SemaphoreType.DMA,
            pltpu.SemaphoreType.DMA,
        ],
    )
    def sc_kernel(uid_hbm, aid_hbm, utab_hbm, atab_hbm, out_hbm,
                  uidx_v, aidx_v, u0, a0, u1, a1, o0, o1, sem0, sem1):
        wid = lax.axis_index("s") * NC + lax.axis_index("c")
        base_row = wid * rows_per_w
        base_idx = base_row * L

        pltpu.sync_copy(uid_hbm.at[pl.ds(base_idx, idx_per_w)], uidx_v)
        pltpu.sync_copy(aid_hbm.at[pl.ds(base_idx, idx_per_w)], aidx_v)

        def start(chunk, u_v, a_v, sem):
            c0 = chunk * IDX
            for k in range(0, IDX, 128):
                n = min(128, IDX - k)
                pltpu.async_copy(
                    utab_hbm.at[uidx_v.at[pl.ds(c0 + k, n)]],
                    u_v.at[pl.ds(k, n)], sem)
                pltpu.async_copy(
                    atab_hbm.at[aidx_v.at[pl.ds(c0 + k, n)]],
                    a_v.at[pl.ds(k, n)], sem)

        def drain(chunk, u_v, a_v, sem):
            c0 = chunk * IDX
            for k in range(0, IDX, 128):
                n = min(128, IDX - k)
                pltpu.make_async_copy(
                    utab_hbm.at[uidx_v.at[pl.ds(c0 + k, n)]],
                    u_v.at[pl.ds(k, n)], sem).wait()
                pltpu.make_async_copy(
                    atab_hbm.at[aidx_v.at[pl.ds(c0 + k, n)]],
                    a_v.at[pl.ds(k, n)], sem).wait()

        def compute(u_v, a_v, o_v):
            @pl.loop(0, CB)
            def _(i):
                r0 = i * L
                for c in range(n_lane_grp):
                    sl = pl.ds(c * LANES, LANES)
                    acc = u_v[r0, sl] * a_v[r0, sl]
                    for l in range(1, L):
                        acc = acc + u_v[r0 + l, sl] * a_v[r0 + l, sl]
                    o_v[i, sl] = acc

        start(0, u0, a0, sem0)

        @pl.loop(0, chunks, step=2)
        def _(g):
            row0 = base_row + g * CB
            start(g + 1, u1, a1, sem1)
            drain(g, u0, a0, sem0)
            compute(u0, a0, o0)
            pltpu.sync_copy(o0, out_hbm.at[pl.ds(row0, CB)])

            @pl.when(g + 2 < chunks)
            def _():
                start(g + 2, u0, a0, sem0)

            drain(g + 1, u1, a1, sem1)
            compute(u1, a1, o1)
            pltpu.sync_copy(o1, out_hbm.at[pl.ds(row0 + CB, CB)])

    return sc_kernel(uid_flat, aid_flat, u_lin, a_lin)


def _tc_head(dot, W, b, B, E):
    BLK = 1024

    def body(d_ref, w_ref, b_ref, o_ref):
        s = jnp.dot(d_ref[...], w_ref[...],
                    preferred_element_type=jnp.float32)
        o_ref[...] = jax.nn.sigmoid(s + b_ref[0, 0])

    return pl.pallas_call(
        body,
        grid=(B // BLK,),
        in_specs=[
            pl.BlockSpec((BLK, E), lambda i: (i, 0)),
            pl.BlockSpec((E, 1), lambda i: (0, 0)),
            pl.BlockSpec((1, 1), lambda i: (0, 0)),
        ],
        out_specs=pl.BlockSpec((BLK, 1), lambda i: (i, 0)),
        out_shape=jax.ShapeDtypeStruct((B, 1), jnp.float32),
    )(dot, W, b.reshape(1, 1))


def kernel(user_id, ad_id, user_table, ad_table, W, b):
    B, L = user_id.shape
    E = user_table.shape[1]
    u_lin, a_lin = _sc_pack(user_table, ad_table)
    dot = _sc_dot(user_id.reshape(-1), ad_id.reshape(-1),
                  u_lin, a_lin, B, L, E)
    return _tc_head(dot, W, b, B, E)


# speedup vs baseline: 1.3606x; 1.3606x over previous
import dataclasses
import functools

import jax
import jax.numpy as jnp
from jax import lax
from jax.experimental import pallas as pl
from jax.experimental.pallas import tpu as pltpu
from jax.experimental.pallas import tpu_sc as plsc


def _sc_params(**kw):
    cp = pltpu.CompilerParams(**kw)
    if "needs_layout_passes" in pltpu.CompilerParams.__dataclass_fields__:
        cp = dataclasses.replace(cp, needs_layout_passes=False)
    return cp

NC = 2
NS = 16
NW = NC * NS
LANES = 16

CB = 16


def _sc_pack(user_table, ad_table):
    Vu, E = user_table.shape
    Va = ad_table.shape[0]
    NBu = Vu // 128
    NBa = Va // 128
    TU = 2 * ((NBu // NW + 2) // 2)
    TA = 2 * ((NBa // NW + 2) // 2)

    ut_t = user_table.T
    at_t = ad_table.T
    ut_last = user_table[Vu - 128:].T
    at_last = ad_table[Va - 128:].T

    mesh = plsc.VectorSubcoreMesh(core_axis_name="c", subcore_axis_name="s")

    @functools.partial(
        pl.kernel,
        mesh=mesh,
        compiler_params=_sc_params(use_tc_tiling_on_sc=True),
        out_type=(jax.ShapeDtypeStruct((Vu // 2, 128), jnp.float32),
                  jax.ShapeDtypeStruct((Va // 2, 128), jnp.float32)),
        scratch_types=[
            pltpu.VMEM((E, 128), jnp.float32),
            pltpu.VMEM((E, 128), jnp.float32),
            pltpu.VMEM((64, 128), jnp.float32),
            pltpu.VMEM((64, 128), jnp.float32),
            pltpu.SemaphoreType.DMA,
            pltpu.SemaphoreType.DMA,
            pltpu.SemaphoreType.DMA,
            pltpu.SemaphoreType.DMA,
        ],
    )
    def pack_kernel(ut_hbm, at_hbm, ulast_hbm, alast_hbm, ou_hbm, oa_hbm,
                    tin0, tin1, ob0, ob1, si0, si1, so0, so1):
        wid = lax.axis_index("s") * NC + lax.axis_index("c")

        iota = lax.iota(jnp.int32, 16)
        half = iota >> 1
        par64 = (iota & 1) * 64
        row_base = [half + g * 8 for g in range(8)]

        def transpose(tin, ob):
            @plsc.parallel_loop(0, E, unroll=8)
            def _(j):
                col = par64 + j
                for g in range(8):
                    v = tin[j, pl.ds(g * 16, 16)]
                    plsc.store_scatter(ob, [row_base[g], col], v)

        def run_table(src_hbm, out_hbm, nb, t_slots):
            def kb_of(t):
                return t * NW + wid

            def start_in(t, tin, sem):
                @pl.when(kb_of(t) < nb)
                def _():
                    pltpu.async_copy(
                        src_hbm.at[:, pl.ds(kb_of(t) * 128, 128)], tin, sem)

            def body(g, t, tin, ob, sem_i, sem_o):
                @pl.when(kb_of(t) < nb)
                def _():
                    pltpu.make_async_copy(
                        src_hbm.at[:, pl.ds(0, 128)], tin, sem_i).wait()

                    @pl.when(g >= 2)
                    def _():
                        pltpu.make_async_copy(
                            out_hbm.at[pl.ds(0, 64)], ob, sem_o).wait()

                    transpose(tin, ob)
                    pltpu.async_copy(
                        ob, out_hbm.at[pl.ds(kb_of(t) * 64, 64)], sem_o)

            start_in(0, tin0, si0)

            @pl.loop(0, t_slots, step=2)
            def _(g):
                start_in(g + 1, tin1, si1)
                body(g, g, tin0, ob0, si0, so0)
                start_in(g + 2, tin0, si0)
                body(g, g + 1, tin1, ob1, si1, so1)

            pltpu.make_async_copy(out_hbm.at[pl.ds(0, 64)], ob0, so0).wait()
            pltpu.make_async_copy(out_hbm.at[pl.ds(0, 64)], ob1, so1).wait()

        run_table(ut_hbm, ou_hbm, NBu, TU)
        run_table(at_hbm, oa_hbm, NBa, TA)

        @pl.when(wid == 4)
        def _():
            pltpu.sync_copy(ulast_hbm, tin0)
            transpose(tin0, ob0)
            pltpu.sync_copy(ob0, ou_hbm.at[pl.ds((Vu - 128) // 2, 64)])

        @pl.when(wid == 13)
        def _():
            pltpu.sync_copy(alast_hbm, tin1)
            transpose(tin1, ob1)
            pltpu.sync_copy(ob1, oa_hbm.at[pl.ds((Va - 128) // 2, 64)])

    ou, oa = pack_kernel(ut_t, at_t, ut_last, at_last)
    return ou.reshape(Vu, E), oa.reshape(Va, E)


def _sc_dot(uid_flat, aid_flat, u_lin, a_lin, B, L, E):
    IDX = CB * L
    rows_per_w = B // NW
    chunks = rows_per_w // CB
    idx_per_w = rows_per_w * L
    assert rows_per_w % CB == 0 and chunks % 2 == 0
    n_lane_grp = E // LANES

    mesh = plsc.VectorSubcoreMesh(core_axis_name="c", subcore_axis_name="s")

    @functools.partial(
        pl.kernel,
        mesh=mesh,
        compiler_params=pltpu.CompilerParams(use_tc_tiling_on_sc=False),
        out_type=jax.ShapeDtypeStruct((B, E), jnp.float32),
        scratch_types=[
            pltpu.VMEM((idx_per_w,), jnp.int32),
            pltpu.VMEM((idx_per_w,), jnp.int32),
            pltpu.VMEM((IDX, E), jnp.float32),
            pltpu.VMEM((IDX, E), jnp.float32),
            pltpu.VMEM((IDX, E), jnp.float32),
            pltpu.VMEM((IDX, E), jnp.float32),
            pltpu.VMEM((CB, E), jnp.float32),
            pltpu.VMEM((CB, E), jnp.float32),
            pltpu.SemaphoreType.DMA,
            pltpu.SemaphoreType.DMA,
        ],
    )
    def sc_kernel(uid_hbm, aid_hbm, utab_hbm, atab_hbm, out_hbm,
                  uidx_v, aidx_v, u0, a0, u1, a1, o0, o1, sem0, sem1):
        wid = lax.axis_index("s") * NC + lax.axis_index("c")
        base_row = wid * rows_per_w
        base_idx = base_row * L

        pltpu.sync_copy(uid_hbm.at[pl.ds(base_idx, idx_per_w)], uidx_v)
        pltpu.sync_copy(aid_hbm.at[pl.ds(base_idx, idx_per_w)], aidx_v)

        def start(chunk, u_v, a_v, sem):
            c0 = chunk * IDX
            for k in range(0, IDX, 128):
                n = min(128, IDX - k)
                pltpu.async_copy(
                    utab_hbm.at[uidx_v.at[pl.ds(c0 + k, n)]],
                    u_v.at[pl.ds(k, n)], sem)
                pltpu.async_copy(
                    atab_hbm.at[aidx_v.at[pl.ds(c0 + k, n)]],
                    a_v.at[pl.ds(k, n)], sem)

        def drain(chunk, u_v, a_v, sem):
            c0 = chunk * IDX
            for k in range(0, IDX, 128):
                n = min(128, IDX - k)
                pltpu.make_async_copy(
                    utab_hbm.at[uidx_v.at[pl.ds(c0 + k, n)]],
                    u_v.at[pl.ds(k, n)], sem).wait()
                pltpu.make_async_copy(
                    atab_hbm.at[aidx_v.at[pl.ds(c0 + k, n)]],
                    a_v.at[pl.ds(k, n)], sem).wait()

        def compute(u_v, a_v, o_v):
            @pl.loop(0, CB)
            def _(i):
                r0 = i * L
                for c in range(n_lane_grp):
                    sl = pl.ds(c * LANES, LANES)
                    acc = u_v[r0, sl] * a_v[r0, sl]
                    for l in range(1, L):
                        acc = acc + u_v[r0 + l, sl] * a_v[r0 + l, sl]
                    o_v[i, sl] = acc

        start(0, u0, a0, sem0)

        @pl.loop(0, chunks, step=2)
        def _(g):
            row0 = base_row + g * CB
            start(g + 1, u1, a1, sem1)
            drain(g, u0, a0, sem0)
            compute(u0, a0, o0)
            pltpu.sync_copy(o0, out_hbm.at[pl.ds(row0, CB)])

            @pl.when(g + 2 < chunks)
            def _():
                start(g + 2, u0, a0, sem0)

            drain(g + 1, u1, a1, sem1)
            compute(u1, a1, o1)
            pltpu.sync_copy(o1, out_hbm.at[pl.ds(row0 + CB, CB)])

    return sc_kernel(uid_flat, aid_flat, u_lin, a_lin)


def _tc_head(dot, W, b, B, E):
    BLK = 1024

    def body(d_ref, w_ref, b_ref, o_ref):
        s = jnp.dot(d_ref[...], w_ref[...],
                    preferred_element_type=jnp.float32)
        o_ref[...] = jax.nn.sigmoid(s + b_ref[0, 0])

    return pl.pallas_call(
        body,
        grid=(B // BLK,),
        in_specs=[
            pl.BlockSpec((BLK, E), lambda i: (i, 0)),
            pl.BlockSpec((E, 1), lambda i: (0, 0)),
            pl.BlockSpec((1, 1), lambda i: (0, 0)),
        ],
        out_specs=pl.BlockSpec((BLK, 1), lambda i: (i, 0)),
        out_shape=jax.ShapeDtypeStruct((B, 1), jnp.float32),
    )(dot, W, b.reshape(1, 1))


def kernel(user_id, ad_id, user_table, ad_table, W, b):
    B, L = user_id.shape
    E = user_table.shape[1]
    u_lin, a_lin = _sc_pack(user_table, ad_table)
    dot = _sc_dot(user_id.reshape(-1), ad_id.reshape(-1),
                  u_lin, a_lin, B, L, E)
    return _tc_head(dot, W, b, B, E)


# speedup vs baseline: 2.0699x; 1.5214x over previous
import dataclasses
import functools

import jax
import jax.numpy as jnp
from jax import lax
from jax.experimental import pallas as pl
from jax.experimental.pallas import tpu as pltpu
from jax.experimental.pallas import tpu_sc as plsc


def _sc_params(**kw):
    cp = pltpu.CompilerParams(**kw)
    if "needs_layout_passes" in pltpu.CompilerParams.__dataclass_fields__:
        cp = dataclasses.replace(cp, needs_layout_passes=False)
    return cp

NC = 2
NS = 16
NW = NC * NS
LANES = 16

CB = 16


def _sc_pack(user_table, ad_table):
    Vu, E = user_table.shape
    Va = ad_table.shape[0]
    NBu = Vu // 128
    NBa = Va // 128
    TU = 2 * ((NBu // NW + 2) // 2)
    TA = 2 * ((NBa // NW + 2) // 2)

    ut_t = user_table.T
    at_t = ad_table.T
    ut_last = user_table[Vu - 128:].T
    at_last = ad_table[Va - 128:].T

    mesh = plsc.VectorSubcoreMesh(core_axis_name="c", subcore_axis_name="s")

    @functools.partial(
        pl.kernel,
        mesh=mesh,
        compiler_params=_sc_params(use_tc_tiling_on_sc=True),
        out_type=(jax.ShapeDtypeStruct((Vu // 2, 128), jnp.float32),
                  jax.ShapeDtypeStruct((Va // 2, 128), jnp.float32)),
        scratch_types=[
            pltpu.VMEM((E, 128), jnp.float32),
            pltpu.VMEM((E, 128), jnp.float32),
            pltpu.VMEM((64, 128), jnp.float32),
            pltpu.VMEM((64, 128), jnp.float32),
            pltpu.VMEM((8768,), jnp.float32),
            pltpu.VMEM((8768,), jnp.float32),
            pltpu.SemaphoreType.DMA,
            pltpu.SemaphoreType.DMA,
            pltpu.SemaphoreType.DMA,
            pltpu.SemaphoreType.DMA,
        ],
    )
    def pack_kernel(ut_hbm, at_hbm, ulast_hbm, alast_hbm, ou_hbm, oa_hbm,
                    tin0, tin1, ob0, ob1, op0, op1, si0, si1, so0, so1):
        wid = lax.axis_index("s") * NC + lax.axis_index("c")

        iota = lax.iota(jnp.int32, 16)
        half = iota >> 1
        base_vec = [(half + g * 8) * 137 + (iota & 1) * 72 for g in range(8)]

        def transpose(tin, obp, ob):
            @plsc.parallel_loop(0, E, unroll=8)
            def _(j):
                for g in range(8):
                    v = tin[j, pl.ds(g * 16, 16)]
                    plsc.store_scatter(obp, [base_vec[g] + j], v)

            @pl.loop(0, 64)
            def _(r):
                r0 = r * 137
                for c in range(8):
                    off = r0 + (c * 16 if c < 4 else 72 + (c - 4) * 16)
                    ob[r, pl.ds(c * 16, 16)] = obp[pl.ds(off, 16)]

        def run_table(src_hbm, out_hbm, nb, t_slots):
            def kb_of(t):
                return t * NW + wid

            def start_in(t, tin, sem):
                @pl.when(kb_of(t) < nb)
                def _():
                    pltpu.async_copy(
                        src_hbm.at[:, pl.ds(kb_of(t) * 128, 128)], tin, sem)

            def body(g, t, tin, ob, sem_i, sem_o):
                @pl.when(kb_of(t) < nb)
                def _():
                    pltpu.make_async_copy(
                        src_hbm.at[:, pl.ds(0, 128)], tin, sem_i).wait()

                    @pl.when(g >= 2)
                    def _():
                        pltpu.make_async_copy(
                            out_hbm.at[pl.ds(0, 64)], ob, sem_o).wait()

                    transpose(tin, op0, ob)
                    pltpu.async_copy(
                        ob, out_hbm.at[pl.ds(kb_of(t) * 64, 64)], sem_o)

            start_in(0, tin0, si0)

            @pl.loop(0, t_slots, step=2)
            def _(g):
                start_in(g + 1, tin1, si1)
                body(g, g, tin0, ob0, si0, so0)
                start_in(g + 2, tin0, si0)
                body(g, g + 1, tin1, ob1, si1, so1)

            pltpu.make_async_copy(out_hbm.at[pl.ds(0, 64)], ob0, so0).wait()
            pltpu.make_async_copy(out_hbm.at[pl.ds(0, 64)], ob1, so1).wait()

        run_table(ut_hbm, ou_hbm, NBu, TU)
        run_table(at_hbm, oa_hbm, NBa, TA)

        @pl.when(wid == 4)
        def _():
            pltpu.sync_copy(ulast_hbm, tin0)
            transpose(tin0, op0, ob0)
            pltpu.sync_copy(ob0, ou_hbm.at[pl.ds((Vu - 128) // 2, 64)])

        @pl.when(wid == 13)
        def _():
            pltpu.sync_copy(alast_hbm, tin1)
            transpose(tin1, op1, ob1)
            pltpu.sync_copy(ob1, oa_hbm.at[pl.ds((Va - 128) // 2, 64)])

    ou, oa = pack_kernel(ut_t, at_t, ut_last, at_last)
    return ou.reshape(Vu, E), oa.reshape(Va, E)


def _sc_dot(uid_flat, aid_flat, u_lin, a_lin, B, L, E):
    IDX = CB * L
    rows_per_w = B // NW
    chunks = rows_per_w // CB
    idx_per_w = rows_per_w * L
    assert rows_per_w % CB == 0 and chunks % 2 == 0
    n_lane_grp = E // LANES

    mesh = plsc.VectorSubcoreMesh(core_axis_name="c", subcore_axis_name="s")

    @functools.partial(
        pl.kernel,
        mesh=mesh,
        compiler_params=pltpu.CompilerParams(use_tc_tiling_on_sc=False),
        out_type=jax.ShapeDtypeStruct((B, E), jnp.float32),
        scratch_types=[
            pltpu.VMEM((idx_per_w,), jnp.int32),
            pltpu.VMEM((idx_per_w,), jnp.int32),
            pltpu.VMEM((IDX, E), jnp.float32),
            pltpu.VMEM((IDX, E), jnp.float32),
            pltpu.VMEM((IDX, E), jnp.float32),
            pltpu.VMEM((IDX, E), jnp.float32),
            pltpu.VMEM((CB, E), jnp.float32),
            pltpu.VMEM((CB, E), jnp.float32),
            pltpu.SemaphoreType.DMA,
            pltpu.SemaphoreType.DMA,
        ],
    )
    def sc_kernel(uid_hbm, aid_hbm, utab_hbm, atab_hbm, out_hbm,
                  uidx_v, aidx_v, u0, a0, u1, a1, o0, o1, sem0, sem1):
        wid = lax.axis_index("s") * NC + lax.axis_index("c")
        base_row = wid * rows_per_w
        base_idx = base_row * L

        pltpu.sync_copy(uid_hbm.at[pl.ds(base_idx, idx_per_w)], uidx_v)
        pltpu.sync_copy(aid_hbm.at[pl.ds(base_idx, idx_per_w)], aidx_v)

        def start(chunk, u_v, a_v, sem):
            c0 = chunk * IDX
            for k in range(0, IDX, 128):
                n = min(128, IDX - k)
                pltpu.async_copy(
                    utab_hbm.at[uidx_v.at[pl.ds(c0 + k, n)]],
                    u_v.at[pl.ds(k, n)], sem)
                pltpu.async_copy(
                    atab_hbm.at[aidx_v.at[pl.ds(c0 + k, n)]],
                    a_v.at[pl.ds(k, n)], sem)

        def drain(chunk, u_v, a_v, sem):
            c0 = chunk * IDX
            for k in range(0, IDX, 128):
                n = min(128, IDX - k)
                pltpu.make_async_copy(
                    utab_hbm.at[uidx_v.at[pl.ds(c0 + k, n)]],
                    u_v.at[pl.ds(k, n)], sem).wait()
                pltpu.make_async_copy(
                    atab_hbm.at[aidx_v.at[pl.ds(c0 + k, n)]],
                    a_v.at[pl.ds(k, n)], sem).wait()

        def compute(u_v, a_v, o_v):
            @pl.loop(0, CB)
            def _(i):
                r0 = i * L
                for c in range(n_lane_grp):
                    sl = pl.ds(c * LANES, LANES)
                    acc = u_v[r0, sl] * a_v[r0, sl]
                    for l in range(1, L):
                        acc = acc + u_v[r0 + l, sl] * a_v[r0 + l, sl]
                    o_v[i, sl] = acc

        start(0, u0, a0, sem0)

        @pl.loop(0, chunks, step=2)
        def _(g):
            row0 = base_row + g * CB
            start(g + 1, u1, a1, sem1)
            drain(g, u0, a0, sem0)
            compute(u0, a0, o0)
            pltpu.sync_copy(o0, out_hbm.at[pl.ds(row0, CB)])

            @pl.when(g + 2 < chunks)
            def _():
                start(g + 2, u0, a0, sem0)

            drain(g + 1, u1, a1, sem1)
            compute(u1, a1, o1)
            pltpu.sync_copy(o1, out_hbm.at[pl.ds(row0 + CB, CB)])

    return sc_kernel(uid_flat, aid_flat, u_lin, a_lin)


def _tc_head(dot, W, b, B, E):
    BLK = 1024

    def body(d_ref, w_ref, b_ref, o_ref):
        s = jnp.dot(d_ref[...], w_ref[...],
                    preferred_element_type=jnp.float32)
        o_ref[...] = jax.nn.sigmoid(s + b_ref[0, 0])

    return pl.pallas_call(
        body,
        grid=(B // BLK,),
        in_specs=[
            pl.BlockSpec((BLK, E), lambda i: (i, 0)),
            pl.BlockSpec((E, 1), lambda i: (0, 0)),
            pl.BlockSpec((1, 1), lambda i: (0, 0)),
        ],
        out_specs=pl.BlockSpec((BLK, 1), lambda i: (i, 0)),
        out_shape=jax.ShapeDtypeStruct((B, 1), jnp.float32),
    )(dot, W, b.reshape(1, 1))


def kernel(user_id, ad_id, user_table, ad_table, W, b):
    B, L = user_id.shape
    E = user_table.shape[1]
    u_lin, a_lin = _sc_pack(user_table, ad_table)
    dot = _sc_dot(user_id.reshape(-1), ad_id.reshape(-1),
                  u_lin, a_lin, B, L, E)
    return _tc_head(dot, W, b, B, E)


# speedup vs baseline: 3.6916x; 1.7834x over previous
import dataclasses
import functools

import jax
import jax.numpy as jnp
from jax import lax
from jax.experimental import pallas as pl
from jax.experimental.pallas import tpu as pltpu
from jax.experimental.pallas import tpu_sc as plsc


def _sc_params(**kw):
    cp = pltpu.CompilerParams(**kw)
    if "needs_layout_passes" in pltpu.CompilerParams.__dataclass_fields__:
        cp = dataclasses.replace(cp, needs_layout_passes=False)
    return cp

NC = 2
NS = 16
NW = NC * NS
LANES = 16

CB = 16


def _sc_pack(user_table, ad_table):
    Vu, E = user_table.shape
    Va = ad_table.shape[0]
    NBu = Vu // 128
    NBa = Va // 128
    TU = 2 * ((NBu // NW + 2) // 2)
    TA = 2 * ((NBa // NW + 2) // 2)

    ut_t = user_table.T
    at_t = ad_table.T
    ut_last = user_table[Vu - 128:].T
    at_last = ad_table[Va - 128:].T

    mesh = plsc.VectorSubcoreMesh(core_axis_name="c", subcore_axis_name="s")

    @functools.partial(
        pl.kernel,
        mesh=mesh,
        compiler_params=_sc_params(use_tc_tiling_on_sc=True),
        out_type=(jax.ShapeDtypeStruct((Vu // 2, 128), jnp.float32),
                  jax.ShapeDtypeStruct((Va // 2, 128), jnp.float32)),
        scratch_types=[
            pltpu.VMEM((E, 128), jnp.float32),
            pltpu.VMEM((E, 128), jnp.float32),
            pltpu.VMEM((64, 128), jnp.float32),
            pltpu.VMEM((64, 128), jnp.float32),
            pltpu.VMEM((8768,), jnp.float32),
            pltpu.VMEM((8768,), jnp.float32),
            pltpu.SemaphoreType.DMA,
            pltpu.SemaphoreType.DMA,
            pltpu.SemaphoreType.DMA,
            pltpu.SemaphoreType.DMA,
        ],
    )
    def pack_kernel(ut_hbm, at_hbm, ulast_hbm, alast_hbm, ou_hbm, oa_hbm,
                    tin0, tin1, ob0, ob1, op0, op1, si0, si1, so0, so1):
        wid = lax.axis_index("s") * NC + lax.axis_index("c")

        iota = lax.iota(jnp.int32, 16)
        half = iota >> 1
        base_vec = [(half + g * 8) * 137 + (iota & 1) * 72 for g in range(8)]

        def transpose(tin, obp, ob):
            @plsc.parallel_loop(0, E, unroll=8)
            def _(j):
                for g in range(8):
                    v = tin[j, pl.ds(g * 16, 16)]
                    plsc.store_scatter(obp, [base_vec[g] + j], v)

            @plsc.parallel_loop(0, 64, unroll=4)
            def _(r):
                r0 = r * 137
                for c in range(8):
                    off = r0 + (c * 16 if c < 4 else 72 + (c - 4) * 16)
                    ob[r, pl.ds(c * 16, 16)] = obp[pl.ds(off, 16)]

        def run_table(src_hbm, out_hbm, nb, t_slots):
            def kb_of(t):
                return t * NW + wid

            def start_in(t, tin, sem):
                @pl.when(kb_of(t) < nb)
                def _():
                    pltpu.async_copy(
                        src_hbm.at[:, pl.ds(kb_of(t) * 128, 128)], tin, sem)

            def body(g, t, tin, ob, sem_i, sem_o):
                @pl.when(kb_of(t) < nb)
                def _():
                    pltpu.make_async_copy(
                        src_hbm.at[:, pl.ds(0, 128)], tin, sem_i).wait()

                    @pl.when(g >= 2)
                    def _():
                        pltpu.make_async_copy(
                            out_hbm.at[pl.ds(0, 64)], ob, sem_o).wait()

                    transpose(tin, op0, ob)
                    pltpu.async_copy(
                        ob, out_hbm.at[pl.ds(kb_of(t) * 64, 64)], sem_o)

            start_in(0, tin0, si0)

            @pl.loop(0, t_slots, step=2)
            def _(g):
                start_in(g + 1, tin1, si1)
                body(g, g, tin0, ob0, si0, so0)
                start_in(g + 2, tin0, si0)
                body(g, g + 1, tin1, ob1, si1, so1)

            pltpu.make_async_copy(out_hbm.at[pl.ds(0, 64)], ob0, so0).wait()
            pltpu.make_async_copy(out_hbm.at[pl.ds(0, 64)], ob1, so1).wait()

        run_table(ut_hbm, ou_hbm, NBu, TU)
        run_table(at_hbm, oa_hbm, NBa, TA)

        @pl.when(wid == 4)
        def _():
            pltpu.sync_copy(ulast_hbm, tin0)
            transpose(tin0, op0, ob0)
            pltpu.sync_copy(ob0, ou_hbm.at[pl.ds((Vu - 128) // 2, 64)])

        @pl.when(wid == 13)
        def _():
            pltpu.sync_copy(alast_hbm, tin1)
            transpose(tin1, op1, ob1)
            pltpu.sync_copy(ob1, oa_hbm.at[pl.ds((Va - 128) // 2, 64)])

    ou, oa = pack_kernel(ut_t, at_t, ut_last, at_last)
    return ou.reshape(Vu, E), oa.reshape(Va, E)


def _sc_dot(uid_flat, aid_flat, u_lin, a_lin, B, L, E):
    IDX = CB * L
    rows_per_w = B // NW
    chunks = rows_per_w // CB
    idx_per_w = rows_per_w * L
    assert rows_per_w % CB == 0 and chunks % 2 == 0
    n_lane_grp = E // LANES

    mesh = plsc.VectorSubcoreMesh(core_axis_name="c", subcore_axis_name="s")

    @functools.partial(
        pl.kernel,
        mesh=mesh,
        compiler_params=pltpu.CompilerParams(use_tc_tiling_on_sc=False),
        out_type=jax.ShapeDtypeStruct((B, E), jnp.float32),
        scratch_types=[
            pltpu.VMEM((idx_per_w,), jnp.int32),
            pltpu.VMEM((idx_per_w,), jnp.int32),
            pltpu.VMEM((IDX, E), jnp.float32),
            pltpu.VMEM((IDX, E), jnp.float32),
            pltpu.VMEM((IDX, E), jnp.float32),
            pltpu.VMEM((IDX, E), jnp.float32),
            pltpu.VMEM((CB, E), jnp.float32),
            pltpu.VMEM((CB, E), jnp.float32),
            pltpu.SemaphoreType.DMA,
            pltpu.SemaphoreType.DMA,
        ],
    )
    def sc_kernel(uid_hbm, aid_hbm, utab_hbm, atab_hbm, out_hbm,
                  uidx_v, aidx_v, u0, a0, u1, a1, o0, o1, sem0, sem1):
        wid = lax.axis_index("s") * NC + lax.axis_index("c")
        base_row = wid * rows_per_w
        base_idx = base_row * L

        pltpu.sync_copy(uid_hbm.at[pl.ds(base_idx, idx_per_w)], uidx_v)
        pltpu.sync_copy(aid_hbm.at[pl.ds(base_idx, idx_per_w)], aidx_v)

        def start(chunk, u_v, a_v, sem):
            c0 = chunk * IDX
            for k in range(0, IDX, 128):
                n = min(128, IDX - k)
                pltpu.async_copy(
                    utab_hbm.at[uidx_v.at[pl.ds(c0 + k, n)]],
                    u_v.at[pl.ds(k, n)], sem)
                pltpu.async_copy(
                    atab_hbm.at[aidx_v.at[pl.ds(c0 + k, n)]],
                    a_v.at[pl.ds(k, n)], sem)

        def drain(chunk, u_v, a_v, sem):
            c0 = chunk * IDX
            for k in range(0, IDX, 128):
                n = min(128, IDX - k)
                pltpu.make_async_copy(
                    utab_hbm.at[uidx_v.at[pl.ds(c0 + k, n)]],
                    u_v.at[pl.ds(k, n)], sem).wait()
                pltpu.make_async_copy(
                    atab_hbm.at[aidx_v.at[pl.ds(c0 + k, n)]],
                    a_v.at[pl.ds(k, n)], sem).wait()

        def compute(u_v, a_v, o_v):
            @pl.loop(0, CB)
            def _(i):
                r0 = i * L
                for c in range(n_lane_grp):
                    sl = pl.ds(c * LANES, LANES)
                    acc = u_v[r0, sl] * a_v[r0, sl]
                    for l in range(1, L):
                        acc = acc + u_v[r0 + l, sl] * a_v[r0 + l, sl]
                    o_v[i, sl] = acc

        start(0, u0, a0, sem0)

        @pl.loop(0, chunks, step=2)
        def _(g):
            row0 = base_row + g * CB
            start(g + 1, u1, a1, sem1)
            drain(g, u0, a0, sem0)
            compute(u0, a0, o0)
            pltpu.sync_copy(o0, out_hbm.at[pl.ds(row0, CB)])

            @pl.when(g + 2 < chunks)
            def _():
                start(g + 2, u0, a0, sem0)

            drain(g + 1, u1, a1, sem1)
            compute(u1, a1, o1)
            pltpu.sync_copy(o1, out_hbm.at[pl.ds(row0 + CB, CB)])

    return sc_kernel(uid_flat, aid_flat, u_lin, a_lin)


def _tc_head(dot, W, b, B, E):
    BLK = 1024

    def body(d_ref, w_ref, b_ref, o_ref):
        s = jnp.dot(d_ref[...], w_ref[...],
                    preferred_element_type=jnp.float32)
        o_ref[...] = jax.nn.sigmoid(s + b_ref[0, 0])

    return pl.pallas_call(
        body,
        grid=(B // BLK,),
        in_specs=[
            pl.BlockSpec((BLK, E), lambda i: (i, 0)),
            pl.BlockSpec((E, 1), lambda i: (0, 0)),
            pl.BlockSpec((1, 1), lambda i: (0, 0)),
        ],
        out_specs=pl.BlockSpec((BLK, 1), lambda i: (i, 0)),
        out_shape=jax.ShapeDtypeStruct((B, 1), jnp.float32),
    )(dot, W, b.reshape(1, 1))


def kernel(user_id, ad_id, user_table, ad_table, W, b):
    B, L = user_id.shape
    E = user_table.shape[1]
    u_lin, a_lin = _sc_pack(user_table, ad_table)
    dot = _sc_dot(user_id.reshape(-1), ad_id.reshape(-1),
                  u_lin, a_lin, B, L, E)
    return _tc_head(dot, W, b, B, E)


# speedup vs baseline: 3.7437x; 1.0141x over previous
import dataclasses
import functools

import jax
import jax.numpy as jnp
from jax import lax
from jax.experimental import pallas as pl
from jax.experimental.pallas import tpu as pltpu
from jax.experimental.pallas import tpu_sc as plsc


def _sc_params(**kw):
    cp = pltpu.CompilerParams(**kw)
    if "needs_layout_passes" in pltpu.CompilerParams.__dataclass_fields__:
        cp = dataclasses.replace(cp, needs_layout_passes=False)
    return cp

NC = 2
NS = 16
NW = NC * NS
LANES = 16

CB = 16


def _sc_pack(user_table, ad_table):
    Vu, E = user_table.shape
    Va = ad_table.shape[0]
    NBu = Vu // 128
    NBa = Va // 128
    TU = 2 * ((NBu // NW + 2) // 2)
    TA = 2 * ((NBa // NW + 2) // 2)

    ut_t = user_table.T
    at_t = ad_table.T
    ut_last = user_table[Vu - 128:].T
    at_last = ad_table[Va - 128:].T

    mesh = plsc.VectorSubcoreMesh(core_axis_name="c", subcore_axis_name="s")

    @functools.partial(
        pl.kernel,
        mesh=mesh,
        compiler_params=_sc_params(use_tc_tiling_on_sc=True),
        out_type=(jax.ShapeDtypeStruct((Vu // 2, 128), jnp.float32),
                  jax.ShapeDtypeStruct((Va // 2, 128), jnp.float32)),
        scratch_types=[
            pltpu.VMEM((E, 128), jnp.float32),
            pltpu.VMEM((E, 128), jnp.float32),
            pltpu.VMEM((64, 128), jnp.float32),
            pltpu.VMEM((64, 128), jnp.float32),
            pltpu.VMEM((8768,), jnp.float32),
            pltpu.VMEM((8768,), jnp.float32),
            pltpu.SemaphoreType.DMA,
            pltpu.SemaphoreType.DMA,
            pltpu.SemaphoreType.DMA,
            pltpu.SemaphoreType.DMA,
        ],
    )
    def pack_kernel(ut_hbm, at_hbm, ulast_hbm, alast_hbm, ou_hbm, oa_hbm,
                    tin0, tin1, ob0, ob1, op0, op1, si0, si1, so0, so1):
        wid = lax.axis_index("s") * NC + lax.axis_index("c")

        iota = lax.iota(jnp.int32, 16)
        half = iota >> 1
        base_vec = [(half + g * 8) * 137 + (iota & 1) * 72 for g in range(8)]

        def transpose(tin, obp, ob):
            @plsc.parallel_loop(0, E, unroll=8)
            def _(j):
                for g in range(8):
                    v = tin[j, pl.ds(g * 16, 16)]
                    plsc.store_scatter(obp, [base_vec[g] + j], v)

            @plsc.parallel_loop(0, 64, unroll=4)
            def _(r):
                r0 = r * 137
                for c in range(8):
                    off = r0 + (c * 16 if c < 4 else 72 + (c - 4) * 16)
                    ob[r, pl.ds(c * 16, 16)] = obp[pl.ds(off, 16)]

        def run_table(src_hbm, out_hbm, nb, t_slots):
            def kb_of(t):
                return t * NW + wid

            def start_in(t, tin, sem):
                @pl.when(kb_of(t) < nb)
                def _():
                    pltpu.async_copy(
                        src_hbm.at[:, pl.ds(kb_of(t) * 128, 128)], tin, sem)

            def body(g, t, tin, ob, sem_i, sem_o):
                @pl.when(kb_of(t) < nb)
                def _():
                    pltpu.make_async_copy(
                        src_hbm.at[:, pl.ds(0, 128)], tin, sem_i).wait()

                    @pl.when(g >= 2)
                    def _():
                        pltpu.make_async_copy(
                            out_hbm.at[pl.ds(0, 64)], ob, sem_o).wait()

                    transpose(tin, op0, ob)
                    pltpu.async_copy(
                        ob, out_hbm.at[pl.ds(kb_of(t) * 64, 64)], sem_o)

            start_in(0, tin0, si0)

            @pl.loop(0, t_slots, step=2)
            def _(g):
                start_in(g + 1, tin1, si1)
                body(g, g, tin0, ob0, si0, so0)
                start_in(g + 2, tin0, si0)
                body(g, g + 1, tin1, ob1, si1, so1)

            pltpu.make_async_copy(out_hbm.at[pl.ds(0, 64)], ob0, so0).wait()
            pltpu.make_async_copy(out_hbm.at[pl.ds(0, 64)], ob1, so1).wait()

        run_table(ut_hbm, ou_hbm, NBu, TU)
        run_table(at_hbm, oa_hbm, NBa, TA)

        @pl.when(wid == 4)
        def _():
            pltpu.sync_copy(ulast_hbm, tin0)
            transpose(tin0, op0, ob0)
            pltpu.sync_copy(ob0, ou_hbm.at[pl.ds((Vu - 128) // 2, 64)])

        @pl.when(wid == 13)
        def _():
            pltpu.sync_copy(alast_hbm, tin1)
            transpose(tin1, op1, ob1)
            pltpu.sync_copy(ob1, oa_hbm.at[pl.ds((Va - 128) // 2, 64)])

    ou, oa = pack_kernel(ut_t, at_t, ut_last, at_last)
    return ou.reshape(Vu, E), oa.reshape(Va, E)


def _sc_dot(uid_flat, aid_flat, u_lin, a_lin, B, L, E):
    IDX = CB * L
    rows_per_w = B // NW
    chunks = rows_per_w // CB
    idx_per_w = rows_per_w * L
    assert rows_per_w % CB == 0 and chunks % 2 == 0
    n_lane_grp = E // LANES

    mesh = plsc.VectorSubcoreMesh(core_axis_name="c", subcore_axis_name="s")

    @functools.partial(
        pl.kernel,
        mesh=mesh,
        compiler_params=pltpu.CompilerParams(use_tc_tiling_on_sc=False),
        out_type=jax.ShapeDtypeStruct((B, E), jnp.float32),
        scratch_types=[
            pltpu.VMEM((idx_per_w,), jnp.int32),
            pltpu.VMEM((idx_per_w,), jnp.int32),
            pltpu.VMEM((IDX, E), jnp.float32),
            pltpu.VMEM((IDX, E), jnp.float32),
            pltpu.VMEM((IDX, E), jnp.float32),
            pltpu.VMEM((IDX, E), jnp.float32),
            pltpu.VMEM((CB, E), jnp.float32),
            pltpu.VMEM((CB, E), jnp.float32),
            pltpu.SemaphoreType.DMA,
            pltpu.SemaphoreType.DMA,
        ],
    )
    def sc_kernel(uid_hbm, aid_hbm, utab_hbm, atab_hbm, out_hbm,
                  uidx_v, aidx_v, u0, a0, u1, a1, o0, o1, sem0, sem1):
        wid = lax.axis_index("s") * NC + lax.axis_index("c")
        base_row = wid * rows_per_w
        base_idx = base_row * L

        pltpu.sync_copy(uid_hbm.at[pl.ds(base_idx, idx_per_w)], uidx_v)
        pltpu.sync_copy(aid_hbm.at[pl.ds(base_idx, idx_per_w)], aidx_v)

        def start(chunk, u_v, a_v, sem):
            c0 = chunk * IDX
            for k in range(0, IDX, 128):
                n = min(128, IDX - k)
                pltpu.async_copy(
                    utab_hbm.at[uidx_v.at[pl.ds(c0 + k, n)]],
                    u_v.at[pl.ds(k, n)], sem)
                pltpu.async_copy(
                    atab_hbm.at[aidx_v.at[pl.ds(c0 + k, n)]],
                    a_v.at[pl.ds(k, n)], sem)

        def drain(chunk, u_v, a_v, sem):
            c0 = chunk * IDX
            for k in range(0, IDX, 128):
                n = min(128, IDX - k)
                pltpu.make_async_copy(
                    utab_hbm.at[uidx_v.at[pl.ds(c0 + k, n)]],
                    u_v.at[pl.ds(k, n)], sem).wait()
                pltpu.make_async_copy(
                    atab_hbm.at[aidx_v.at[pl.ds(c0 + k, n)]],
                    a_v.at[pl.ds(k, n)], sem).wait()

        def compute(u_v, a_v, o_v):
            @pl.loop(0, CB)
            def _(i):
                r0 = i * L
                for c in range(n_lane_grp):
                    sl = pl.ds(c * LANES, LANES)
                    acc = u_v[r0, sl] * a_v[r0, sl]
                    for l in range(1, L):
                        acc = acc + u_v[r0 + l, sl] * a_v[r0 + l, sl]
                    o_v[i, sl] = acc

        start(0, u0, a0, sem0)

        @pl.loop(0, chunks, step=2)
        def _(g):
            row0 = base_row + g * CB
            start(g + 1, u1, a1, sem1)
            drain(g, u0, a0, sem0)
            compute(u0, a0, o0)
            pltpu.sync_copy(o0, out_hbm.at[pl.ds(row0, CB)])

            @pl.when(g + 2 < chunks)
            def _():
                start(g + 2, u0, a0, sem0)

            drain(g + 1, u1, a1, sem1)
            compute(u1, a1, o1)
            pltpu.sync_copy(o1, out_hbm.at[pl.ds(row0 + CB, CB)])

    return sc_kernel(uid_flat, aid_flat, u_lin, a_lin)


def _tc_head(dot, W, b, B, E):
    BLK = 4096

    def body(d_ref, w_ref, b_ref, o_ref):
        s = jnp.dot(d_ref[...], w_ref[...],
                    preferred_element_type=jnp.float32)
        o_ref[...] = jax.nn.sigmoid(s + b_ref[0, 0])

    return pl.pallas_call(
        body,
        grid=(B // BLK,),
        in_specs=[
            pl.BlockSpec((BLK, E), lambda i: (i, 0)),
            pl.BlockSpec((E, 1), lambda i: (0, 0)),
            pl.BlockSpec((1, 1), lambda i: (0, 0)),
        ],
        out_specs=pl.BlockSpec((BLK, 1), lambda i: (i, 0)),
        out_shape=jax.ShapeDtypeStruct((B, 1), jnp.float32),
    )(dot, W, b.reshape(1, 1))


def kernel(user_id, ad_id, user_table, ad_table, W, b):
    B, L = user_id.shape
    E = user_table.shape[1]
    u_lin, a_lin = _sc_pack(user_table, ad_table)
    dot = _sc_dot(user_id.reshape(-1), ad_id.reshape(-1),
                  u_lin, a_lin, B, L, E)
    return _tc_head(dot, W, b, B, E)


# speedup vs baseline: 4.2188x; 1.1269x over previous
import dataclasses
import functools

import jax
import jax.numpy as jnp
from jax import lax
from jax.experimental import pallas as pl
from jax.experimental.pallas import tpu as pltpu
from jax.experimental.pallas import tpu_sc as plsc


def _sc_params(**kw):
    cp = pltpu.CompilerParams(**kw)
    if "needs_layout_passes" in pltpu.CompilerParams.__dataclass_fields__:
        cp = dataclasses.replace(cp, needs_layout_passes=False)
    return cp

NC = 2
NS = 16
NW = NC * NS
LANES = 16

CB = 16


def _sc_pack(user_table, ad_table):
    Vu, E = user_table.shape
    Va = ad_table.shape[0]
    NBu = Vu // 256
    NBa = Va // 128
    TU = 2 * ((NBu // NW + 2) // 2)
    TA = 2 * ((NBa // NW + 2) // 2)

    ut_t = user_table.T
    at_t = ad_table.T
    ut_last = user_table[Vu - 128:].T
    at_last = ad_table[Va - 128:].T

    mesh = plsc.VectorSubcoreMesh(core_axis_name="c", subcore_axis_name="s")

    @functools.partial(
        pl.kernel,
        mesh=mesh,
        compiler_params=_sc_params(use_tc_tiling_on_sc=True),
        out_type=(jax.ShapeDtypeStruct((Vu // 2, 128), jnp.float32),
                  jax.ShapeDtypeStruct((Va // 2, 128), jnp.float32)),
        scratch_types=[
            pltpu.VMEM((E, 256), jnp.float32),
            pltpu.VMEM((E, 256), jnp.float32),
            pltpu.VMEM((128, 128), jnp.float32),
            pltpu.VMEM((128, 128), jnp.float32),
            pltpu.VMEM((17536,), jnp.float32),
            pltpu.VMEM((17536,), jnp.float32),
            pltpu.SemaphoreType.DMA,
            pltpu.SemaphoreType.DMA,
            pltpu.SemaphoreType.DMA,
            pltpu.SemaphoreType.DMA,
        ],
    )
    def pack_kernel(ut_hbm, at_hbm, ulast_hbm, alast_hbm, ou_hbm, oa_hbm,
                    tin0, tin1, ob0, ob1, op0, op1, si0, si1, so0, so1):
        wid = lax.axis_index("s") * NC + lax.axis_index("c")

        iota = lax.iota(jnp.int32, 16)
        half = iota >> 1
        base_vec = [(half + g * 8) * 137 + (iota & 1) * 72
                    for g in range(16)]

        def transpose(tin, obp, ob, ng):
            @plsc.parallel_loop(0, E, unroll=8)
            def _(j):
                for g in range(ng):
                    v = tin[j, pl.ds(g * 16, 16)]
                    plsc.store_scatter(obp, [base_vec[g] + j], v)

            @plsc.parallel_loop(0, ng * 8, unroll=4)
            def _(r):
                r0 = r * 137
                for c in range(8):
                    off = r0 + (c * 16 if c < 4 else 72 + (c - 4) * 16)
                    ob[r, pl.ds(c * 16, 16)] = obp[pl.ds(off, 16)]

        def run_table(src_hbm, out_hbm, nb, t_slots, bu):
            hu = bu // 2

            def kb_of(t):
                return t * NW + wid

            def start_in(t, tin, sem):
                @pl.when(kb_of(t) < nb)
                def _():
                    pltpu.async_copy(
                        src_hbm.at[:, pl.ds(kb_of(t) * bu, bu)],
                        tin.at[:, pl.ds(0, bu)], sem)

            def body(g, t, tin, ob, sem_i, sem_o):
                @pl.when(kb_of(t) < nb)
                def _():
                    pltpu.make_async_copy(
                        src_hbm.at[:, pl.ds(0, bu)],
                        tin.at[:, pl.ds(0, bu)], sem_i).wait()

                    @pl.when(g >= 2)
                    def _():
                        pltpu.make_async_copy(
                            out_hbm.at[pl.ds(0, hu)],
                            ob.at[pl.ds(0, hu)], sem_o).wait()

                    transpose(tin, op0, ob, bu // 16)
                    pltpu.async_copy(
                        ob.at[pl.ds(0, hu)],
                        out_hbm.at[pl.ds(kb_of(t) * hu, hu)], sem_o)

            start_in(0, tin0, si0)

            @pl.loop(0, t_slots, step=2)
            def _(g):
                start_in(g + 1, tin1, si1)
                body(g, g, tin0, ob0, si0, so0)
                start_in(g + 2, tin0, si0)
                body(g, g + 1, tin1, ob1, si1, so1)

            pltpu.make_async_copy(
                out_hbm.at[pl.ds(0, hu)], ob0.at[pl.ds(0, hu)], so0).wait()
            pltpu.make_async_copy(
                out_hbm.at[pl.ds(0, hu)], ob1.at[pl.ds(0, hu)], so1).wait()

        run_table(ut_hbm, ou_hbm, NBu, TU, 256)
        run_table(at_hbm, oa_hbm, NBa, TA, 128)

        @pl.when(wid == 4)
        def _():
            pltpu.sync_copy(ulast_hbm, tin0.at[:, pl.ds(0, 128)])
            transpose(tin0, op0, ob0, 8)
            pltpu.sync_copy(ob0.at[pl.ds(0, 64)],
                            ou_hbm.at[pl.ds((Vu - 128) // 2, 64)])

        @pl.when(wid == 13)
        def _():
            pltpu.sync_copy(alast_hbm, tin1.at[:, pl.ds(0, 128)])
            transpose(tin1, op1, ob1, 8)
            pltpu.sync_copy(ob1.at[pl.ds(0, 64)],
                            oa_hbm.at[pl.ds((Va - 128) // 2, 64)])

    ou, oa = pack_kernel(ut_t, at_t, ut_last, at_last)
    return ou.reshape(Vu, E), oa.reshape(Va, E)


def _sc_dot(uid_flat, aid_flat, u_lin, a_lin, B, L, E):
    IDX = CB * L
    rows_per_w = B // NW
    chunks = rows_per_w // CB
    idx_per_w = rows_per_w * L
    assert rows_per_w % CB == 0 and chunks % 2 == 0
    n_lane_grp = E // LANES

    mesh = plsc.VectorSubcoreMesh(core_axis_name="c", subcore_axis_name="s")

    @functools.partial(
        pl.kernel,
        mesh=mesh,
        compiler_params=pltpu.CompilerParams(use_tc_tiling_on_sc=False),
        out_type=jax.ShapeDtypeStruct((B, E), jnp.float32),
        scratch_types=[
            pltpu.VMEM((idx_per_w,), jnp.int32),
            pltpu.VMEM((idx_per_w,), jnp.int32),
            pltpu.VMEM((IDX, E), jnp.float32),
            pltpu.VMEM((IDX, E), jnp.float32),
            pltpu.VMEM((IDX, E), jnp.float32),
            pltpu.VMEM((IDX, E), jnp.float32),
            pltpu.VMEM((CB, E), jnp.float32),
            pltpu.VMEM((CB, E), jnp.float32),
            pltpu.SemaphoreType.DMA,
            pltpu.SemaphoreType.DMA,
        ],
    )
    def sc_kernel(uid_hbm, aid_hbm, utab_hbm, atab_hbm, out_hbm,
                  uidx_v, aidx_v, u0, a0, u1, a1, o0, o1, sem0, sem1):
        wid = lax.axis_index("s") * NC + lax.axis_index("c")
        base_row = wid * rows_per_w
        base_idx = base_row * L

        pltpu.sync_copy(uid_hbm.at[pl.ds(base_idx, idx_per_w)], uidx_v)
        pltpu.sync_copy(aid_hbm.at[pl.ds(base_idx, idx_per_w)], aidx_v)

        def start(chunk, u_v, a_v, sem):
            c0 = chunk * IDX
            for k in range(0, IDX, 128):
                n = min(128, IDX - k)
                pltpu.async_copy(
                    utab_hbm.at[uidx_v.at[pl.ds(c0 + k, n)]],
                    u_v.at[pl.ds(k, n)], sem)
                pltpu.async_copy(
                    atab_hbm.at[aidx_v.at[pl.ds(c0 + k, n)]],
                    a_v.at[pl.ds(k, n)], sem)

        def drain(chunk, u_v, a_v, sem):
            c0 = chunk * IDX
            for k in range(0, IDX, 128):
                n = min(128, IDX - k)
                pltpu.make_async_copy(
                    utab_hbm.at[uidx_v.at[pl.ds(c0 + k, n)]],
                    u_v.at[pl.ds(k, n)], sem).wait()
                pltpu.make_async_copy(
                    atab_hbm.at[aidx_v.at[pl.ds(c0 + k, n)]],
                    a_v.at[pl.ds(k, n)], sem).wait()

        def compute(u_v, a_v, o_v):
            @pl.loop(0, CB)
            def _(i):
                r0 = i * L
                for c in range(n_lane_grp):
                    sl = pl.ds(c * LANES, LANES)
                    acc = u_v[r0, sl] * a_v[r0, sl]
                    for l in range(1, L):
                        acc = acc + u_v[r0 + l, sl] * a_v[r0 + l, sl]
                    o_v[i, sl] = acc

        start(0, u0, a0, sem0)

        @pl.loop(0, chunks, step=2)
        def _(g):
            row0 = base_row + g * CB
            start(g + 1, u1, a1, sem1)
            drain(g, u0, a0, sem0)
            compute(u0, a0, o0)
            pltpu.sync_copy(o0, out_hbm.at[pl.ds(row0, CB)])

            @pl.when(g + 2 < chunks)
            def _():
                start(g + 2, u0, a0, sem0)

            drain(g + 1, u1, a1, sem1)
            compute(u1, a1, o1)
            pltpu.sync_copy(o1, out_hbm.at[pl.ds(row0 + CB, CB)])

    return sc_kernel(uid_flat, aid_flat, u_lin, a_lin)


def _tc_head(dot, W, b, B, E):
    BLK = 4096

    def body(d_ref, w_ref, b_ref, o_ref):
        s = jnp.dot(d_ref[...], w_ref[...],
                    preferred_element_type=jnp.float32)
        o_ref[...] = jax.nn.sigmoid(s + b_ref[0, 0])

    return pl.pallas_call(
        body,
        grid=(B // BLK,),
        in_specs=[
            pl.BlockSpec((BLK, E), lambda i: (i, 0)),
            pl.BlockSpec((E, 1), lambda i: (0, 0)),
            pl.BlockSpec((1, 1), lambda i: (0, 0)),
        ],
        out_specs=pl.BlockSpec((BLK, 1), lambda i: (i, 0)),
        out_shape=jax.ShapeDtypeStruct((B, 1), jnp.float32),
    )(dot, W, b.reshape(1, 1))


def kernel(user_id, ad_id, user_table, ad_table, W, b):
    B, L = user_id.shape
    E = user_table.shape[1]
    u_lin, a_lin = _sc_pack(user_table, ad_table)
    dot = _sc_dot(user_id.reshape(-1), ad_id.reshape(-1),
                  u_lin, a_lin, B, L, E)
    return _tc_head(dot, W, b, B, E)
